# baseline (speedup 1.0000x reference)
import jax
import jax.numpy as jnp
from jax.experimental import pallas as pl

_K = 8


def _split(x):
    hi_f32 = jax.lax.bitcast_convert_type(
        jax.lax.bitcast_convert_type(x, jnp.uint32) & jnp.uint32(0xFFFF0000),
        jnp.float32,
    )
    lo = (x - hi_f32).astype(jnp.bfloat16)
    return hi_f32.astype(jnp.bfloat16), lo


def _chamfer_kernel(x1_ref, x2_ref, dist1_ref, dist2_ref):
    m_idx = pl.program_id(1)

    x1 = x1_ref[0]
    x2 = x2_ref[0]

    h1, l1 = _split(x1)
    h2, l2 = _split(x2)

    dims = (((1,), (0,)), ((), ()))
    d = jax.lax.dot_general(l1, l2, dims, preferred_element_type=jnp.float32)
    d += jax.lax.dot_general(l1, h2, dims, preferred_element_type=jnp.float32)
    d += jax.lax.dot_general(h1, l2, dims, preferred_element_type=jnp.float32)
    d += jax.lax.dot_general(h1, h2, dims, preferred_element_type=jnp.float32)

    tile_min1 = jnp.min(d, axis=1)
    dist2_ref[0, 0] = jnp.min(d, axis=0)

    @pl.when(m_idx == 0)
    def _init():
        dist1_ref[0, 0] = tile_min1

    @pl.when(m_idx != 0)
    def _acc():
        dist1_ref[0, 0] = jnp.minimum(dist1_ref[0, 0], tile_min1)


def kernel(input1, input2):
    B, N, D = input1.shape
    M = input2.shape[1]
    TM = 1024

    sq1 = jnp.sum(input1 * input1, axis=-1, keepdims=True)
    sq2 = jnp.sum(input2 * input2, axis=-1, keepdims=True)
    ones1 = jnp.ones((B, N, 1), jnp.float32)
    ones2 = jnp.ones((B, M, 1), jnp.float32)
    pad1 = jnp.zeros((B, N, _K - D - 2), jnp.float32)
    pad2 = jnp.zeros((B, M, _K - D - 2), jnp.float32)

    x1aug = jnp.concatenate([-2.0 * input1, sq1, ones1, pad1], axis=-1)
    x2aug = jnp.concatenate([input2, ones2, sq2, pad2], axis=-1)
    x2augT = jnp.transpose(x2aug, (0, 2, 1))

    dist1, dist2 = pl.pallas_call(
        _chamfer_kernel,
        grid=(B, M // TM),
        in_specs=[
            pl.BlockSpec((1, N, _K), lambda b, m: (b, 0, 0)),
            pl.BlockSpec((1, _K, TM), lambda b, m: (b, 0, m)),
        ],
        out_specs=[
            pl.BlockSpec((1, 1, N), lambda b, m: (b, 0, 0)),
            pl.BlockSpec((1, 1, TM), lambda b, m: (b, 0, m)),
        ],
        out_shape=[
            jax.ShapeDtypeStruct((B, 1, N), jnp.float32),
            jax.ShapeDtypeStruct((B, 1, M), jnp.float32),
        ],
    )(x1aug, x2augT)

    return dist1[:, 0, :], dist2[:, 0, :]


# speedup vs baseline: 1.9895x; 1.9895x over previous
import jax
import jax.numpy as jnp
from jax.experimental import pallas as pl


def _chamfer_kernel(x1_ref, x2_ref, dist1_ref, dist2_ref):
    m_idx = pl.program_id(1)

    x1 = x1_ref[0]
    x2 = x2_ref[0]

    sq1 = jnp.sum(x1 * x1, axis=0)
    sq2 = jnp.sum(x2 * x2, axis=0)

    cross = jax.lax.dot_general(
        x1, x2, (((0,), (0,)), ((), ())), preferred_element_type=jnp.float32
    )

    d = sq1[:, None] + sq2[None, :] - 2.0 * cross

    tile_min1 = jnp.min(d, axis=1)
    dist2_ref[0, 0] = jnp.min(d, axis=0)

    @pl.when(m_idx == 0)
    def _init():
        dist1_ref[0, 0] = tile_min1

    @pl.when(m_idx != 0)
    def _acc():
        dist1_ref[0, 0] = jnp.minimum(dist1_ref[0, 0], tile_min1)


def kernel(input1, input2):
    B, N, _ = input1.shape
    M = input2.shape[1]
    TM = 1024

    x1t = jnp.transpose(input1, (0, 2, 1))
    x2t = jnp.transpose(input2, (0, 2, 1))

    dist1, dist2 = pl.pallas_call(
        _chamfer_kernel,
        grid=(B, M // TM),
        in_specs=[
            pl.BlockSpec((1, 3, N), lambda b, m: (b, 0, 0)),
            pl.BlockSpec((1, 3, TM), lambda b, m: (b, 0, m)),
        ],
        out_specs=[
            pl.BlockSpec((1, 1, N), lambda b, m: (b, 0, 0)),
            pl.BlockSpec((1, 1, TM), lambda b, m: (b, 0, m)),
        ],
        out_shape=[
            jax.ShapeDtypeStruct((B, 1, N), jnp.float32),
            jax.ShapeDtypeStruct((B, 1, M), jnp.float32),
        ],
    )(x1t, x2t)

    return dist1[:, 0, :], dist2[:, 0, :]
